# baseline (speedup 1.0000x reference)
import functools

import jax
import jax.numpy as jnp
from jax import lax
from jax.experimental import pallas as pl
from jax.experimental.pallas import tpu as pltpu
from jax.experimental.pallas import tpu_sc as plsc

_BLOCK = 64


@functools.lru_cache(maxsize=None)
def _make_hist_kernel(n_blocks, vocab):
    info = plsc.get_sparse_core_info()
    nc, ns, lanes = info.num_cores, info.num_subcores, info.num_lanes
    nw = nc * ns
    assert n_blocks % nw == 0
    bpw = n_blocks // nw
    assert bpw == lanes, "one block per vreg lane"
    vpad = ((vocab + lanes - 1) // lanes) * lanes

    @functools.partial(
        pl.kernel,
        mesh=plsc.VectorSubcoreMesh(core_axis_name="c", subcore_axis_name="s"),
        compiler_params=pltpu.CompilerParams(
            needs_layout_passes=False, use_tc_tiling_on_sc=False
        ),
        out_type=jax.ShapeDtypeStruct((n_blocks, vocab), jnp.float32),
        scratch_types=[
            pltpu.VMEM((bpw * _BLOCK,), jnp.int32),
            pltpu.VMEM((bpw * vpad,), jnp.float32),
            pltpu.SemaphoreType.DMA,
        ],
    )
    def hist_k(tok_hbm, hist_hbm, tok_v, acc_v, sem):
        wid = lax.axis_index("s") * nc + lax.axis_index("c")
        base_tok = wid * (bpw * _BLOCK)
        pltpu.sync_copy(tok_hbm.at[pl.ds(base_tok, bpw * _BLOCK)], tok_v)

        zeros = jnp.zeros((lanes,), jnp.float32)

        def zero_body(i, carry):
            acc_v[pl.ds(i * lanes, lanes)] = zeros
            return carry

        lax.fori_loop(0, (bpw * vpad) // lanes, zero_body, None)

        rows = lax.iota(jnp.int32, lanes)
        col_base = rows * _BLOCK
        row_base = rows * vpad
        ones = jnp.ones((lanes,), jnp.float32)
        for t in range(_BLOCK):
            tok = plsc.load_gather(tok_v, [col_base + t])
            plsc.addupdate_scatter(acc_v, [row_base + tok], ones)

        base_row = wid * bpw
        copies = [
            pltpu.async_copy(
                acc_v.at[pl.ds(r * vpad, vocab)], hist_hbm.at[base_row + r], sem
            )
            for r in range(bpw)
        ]
        for c in copies:
            c.wait()

    return hist_k


def kernel(tokens, cat_embed_f, num_embed_f):
    B, L = tokens.shape
    vocab = num_embed_f.shape[0]
    n_blocks = (B * L) // _BLOCK
    hist_flat = _make_hist_kernel(n_blocks, vocab)(tokens.reshape(-1))
    hist = hist_flat.reshape(B, L // _BLOCK, vocab)
    cat_ids = tokens[:, ::_BLOCK]
    new_tokens = jnp.concatenate([cat_ids, tokens], axis=1)
    return (new_tokens, cat_ids, hist)


# speedup vs baseline: 1.0636x; 1.0636x over previous
import functools

import jax
import jax.numpy as jnp
from jax import lax
from jax.experimental import pallas as pl
from jax.experimental.pallas import tpu as pltpu
from jax.experimental.pallas import tpu_sc as plsc

_BLOCK = 64


@functools.lru_cache(maxsize=None)
def _make_hist_kernel(n_blocks, vocab):
    info = plsc.get_sparse_core_info()
    nc, ns, lanes = info.num_cores, info.num_subcores, info.num_lanes
    nw = nc * ns
    assert n_blocks % nw == 0
    bpw = n_blocks // nw
    assert bpw == lanes, "one block per vreg lane"
    acc_words = bpw * vocab
    assert acc_words % lanes == 0 and (bpw * vocab) % 8 == 0

    @functools.partial(
        pl.kernel,
        mesh=plsc.VectorSubcoreMesh(core_axis_name="c", subcore_axis_name="s"),
        compiler_params=pltpu.CompilerParams(
            needs_layout_passes=False, use_tc_tiling_on_sc=False
        ),
        out_type=jax.ShapeDtypeStruct((n_blocks * vocab,), jnp.float32),
        scratch_types=[
            pltpu.VMEM((bpw * _BLOCK,), jnp.int32),
            pltpu.VMEM((acc_words,), jnp.float32),
            pltpu.SemaphoreType.DMA,
        ],
    )
    def hist_k(tok_hbm, hist_hbm, tok_v, acc_v, sem):
        wid = lax.axis_index("s") * nc + lax.axis_index("c")
        base_tok = wid * (bpw * _BLOCK)
        pltpu.sync_copy(tok_hbm.at[pl.ds(base_tok, bpw * _BLOCK)], tok_v)

        zeros = jnp.zeros((lanes,), jnp.float32)
        for i in range(acc_words // lanes):
            acc_v[pl.ds(i * lanes, lanes)] = zeros

        rows = lax.iota(jnp.int32, lanes)
        col_base = rows * _BLOCK
        row_base = rows * vocab
        ones = jnp.ones((lanes,), jnp.float32)
        for t in range(_BLOCK):
            tok = plsc.load_gather(tok_v, [col_base + t])
            plsc.addupdate_scatter(acc_v, [row_base + tok], ones)

        pltpu.sync_copy(acc_v, hist_hbm.at[pl.ds(wid * acc_words, acc_words)])

    return hist_k


def kernel(tokens, cat_embed_f, num_embed_f):
    B, L = tokens.shape
    vocab = num_embed_f.shape[0]
    n_blocks = (B * L) // _BLOCK
    hist_flat = _make_hist_kernel(n_blocks, vocab)(tokens.reshape(-1))
    hist = hist_flat.reshape(B, L // _BLOCK, vocab)
    cat_ids = tokens[:, ::_BLOCK]
    new_tokens = jnp.concatenate([cat_ids, tokens], axis=1)
    return (new_tokens, cat_ids, hist)


# speedup vs baseline: 1.1556x; 1.0865x over previous
import functools

import jax
import jax.numpy as jnp
from jax import lax
from jax.experimental import pallas as pl
from jax.experimental.pallas import tpu as pltpu
from jax.experimental.pallas import tpu_sc as plsc

_BLOCK = 64


@functools.lru_cache(maxsize=None)
def _make_hist_kernel(n_blocks, vocab):
    info = plsc.get_sparse_core_info()
    nc, ns, lanes = info.num_cores, info.num_subcores, info.num_lanes
    nw = nc * ns
    assert n_blocks % nw == 0
    bpw = n_blocks // nw
    assert bpw == lanes, "one block per vreg lane"
    acc_words = bpw * vocab
    assert acc_words % lanes == 0 and (bpw * vocab) % 8 == 0

    @functools.partial(
        pl.kernel,
        mesh=plsc.VectorSubcoreMesh(core_axis_name="c", subcore_axis_name="s"),
        compiler_params=pltpu.CompilerParams(
            needs_layout_passes=False, use_tc_tiling_on_sc=False
        ),
        out_type=jax.ShapeDtypeStruct((n_blocks * vocab,), jnp.float32),
        scratch_types=[
            pltpu.VMEM((bpw * _BLOCK,), jnp.int32),
            pltpu.VMEM((acc_words,), jnp.float32),
            pltpu.SemaphoreType.DMA,
        ],
    )
    def hist_k(tok_hbm, hist_hbm, tok_v, acc_v, sem):
        wid = lax.axis_index("s") * nc + lax.axis_index("c")
        base_tok = wid * (bpw * _BLOCK)
        pltpu.sync_copy(tok_hbm.at[pl.ds(base_tok, bpw * _BLOCK)], tok_v)

        zeros = jnp.zeros((lanes,), jnp.float32)
        zunroll = 16
        n_zchunks = acc_words // lanes

        def zero_body(i, carry):
            for j in range(zunroll):
                acc_v[pl.ds((i * zunroll + j) * lanes, lanes)] = zeros
            return carry

        lax.fori_loop(0, n_zchunks // zunroll, zero_body, None)
        for i in range(n_zchunks - n_zchunks % zunroll, n_zchunks):
            acc_v[pl.ds(i * lanes, lanes)] = zeros

        rows = lax.iota(jnp.int32, lanes)
        col_base = rows * _BLOCK
        row_base = rows * vocab
        ones = jnp.ones((lanes,), jnp.float32)
        sunroll = 4

        def scat_body(i, carry):
            t0 = i * sunroll
            for j in range(sunroll):
                tok = plsc.load_gather(tok_v, [col_base + (t0 + j)])
                plsc.addupdate_scatter(acc_v, [row_base + tok], ones)
            return carry

        lax.fori_loop(0, _BLOCK // sunroll, scat_body, None)

        pltpu.sync_copy(acc_v, hist_hbm.at[pl.ds(wid * acc_words, acc_words)])

    return hist_k


def kernel(tokens, cat_embed_f, num_embed_f):
    B, L = tokens.shape
    vocab = num_embed_f.shape[0]
    n_blocks = (B * L) // _BLOCK
    hist_flat = _make_hist_kernel(n_blocks, vocab)(tokens.reshape(-1))
    hist = hist_flat.reshape(B, L // _BLOCK, vocab)
    cat_ids = tokens[:, ::_BLOCK]
    new_tokens = jnp.concatenate([cat_ids, tokens], axis=1)
    return (new_tokens, cat_ids, hist)


# speedup vs baseline: 1.3386x; 1.1584x over previous
import functools

import jax
import jax.numpy as jnp
from jax import lax
from jax.experimental import pallas as pl
from jax.experimental.pallas import tpu as pltpu
from jax.experimental.pallas import tpu_sc as plsc

_BLOCK = 64


@functools.lru_cache(maxsize=None)
def _make_hist_kernel(n_batch, n_blocks, vocab):
    info = plsc.get_sparse_core_info()
    nc, ns, lanes = info.num_cores, info.num_subcores, info.num_lanes
    nw = nc * ns
    total_blocks = n_batch * n_blocks
    assert total_blocks % nw == 0
    bpw = total_blocks // nw
    assert bpw == lanes == 16, "one block per vreg lane"
    assert nw % n_batch == 0 and nw // n_batch == 2, "two workers per batch"
    vtiles = (vocab + 127) // 128

    @functools.partial(
        pl.kernel,
        mesh=plsc.VectorSubcoreMesh(core_axis_name="c", subcore_axis_name="s"),
        compiler_params=pltpu.CompilerParams(needs_layout_passes=False),
        out_type=jax.ShapeDtypeStruct(
            (n_batch, n_blocks, vtiles * 128), jnp.float32
        ),
        scratch_types=[
            pltpu.VMEM((bpw * _BLOCK,), jnp.int32),
            pltpu.VMEM((2, vtiles, 8, 128), jnp.float32),
            pltpu.SemaphoreType.DMA,
        ],
    )
    def hist_k(tok_hbm, hist_hbm, tok_v, acc_v, sem):
        wid = lax.axis_index("s") * nc + lax.axis_index("c")
        b = wid // 2
        h = wid % 2
        pltpu.sync_copy(tok_hbm.at[pl.ds(wid * (bpw * _BLOCK), bpw * _BLOCK)], tok_v)

        zeros = jnp.zeros((lanes,), jnp.float32)

        def zero_body(i, carry):
            c = i // 8
            r = i % 8
            for g in range(2):
                for j in range(128 // lanes):
                    acc_v[g, c, r, pl.ds(j * lanes, lanes)] = zeros
            return carry

        lax.fori_loop(0, vtiles * 8, zero_body, None)

        lane = lax.iota(jnp.int32, lanes)
        col_base = lane * _BLOCK
        g_idx = lane >> 3
        r_idx = lane & 7
        ones = jnp.ones((lanes,), jnp.float32)
        sunroll = 4

        def scat_body(i, carry):
            t0 = i * sunroll
            for j in range(sunroll):
                tok = plsc.load_gather(tok_v, [col_base + (t0 + j)])
                plsc.addupdate_scatter(
                    acc_v, [g_idx, tok >> 7, r_idx, tok & 127], ones
                )
            return carry

        lax.fori_loop(0, _BLOCK // sunroll, scat_body, None)

        copies = []
        for g in range(2):
            for c in range(vtiles):
                copies.append(
                    pltpu.async_copy(
                        acc_v.at[g, c],
                        hist_hbm.at[b, pl.ds(h * 16 + g * 8, 8), pl.ds(c * 128, 128)],
                        sem,
                    )
                )
        for cp in copies:
            cp.wait()

    return hist_k


def kernel(tokens, cat_embed_f, num_embed_f):
    B, L = tokens.shape
    vocab = num_embed_f.shape[0]
    n_blocks = L // _BLOCK
    hist_padded = _make_hist_kernel(B, n_blocks, vocab)(tokens.reshape(-1))
    hist = hist_padded[:, :, :vocab]
    cat_ids = tokens[:, ::_BLOCK]
    new_tokens = jnp.concatenate([cat_ids, tokens], axis=1)
    return (new_tokens, cat_ids, hist)


# speedup vs baseline: 1.3721x; 1.0250x over previous
import functools

import jax
import jax.numpy as jnp
from jax import lax
from jax.experimental import pallas as pl
from jax.experimental.pallas import tpu as pltpu
from jax.experimental.pallas import tpu_sc as plsc

_BLOCK = 64


@functools.lru_cache(maxsize=None)
def _make_hist_kernel(n_batch, n_blocks, vocab):
    info = plsc.get_sparse_core_info()
    nc, ns, lanes = info.num_cores, info.num_subcores, info.num_lanes
    nw = nc * ns
    total_blocks = n_batch * n_blocks
    assert total_blocks % nw == 0
    bpw = total_blocks // nw
    assert bpw == lanes == 16, "one block per vreg lane"
    assert nw % n_batch == 0 and nw // n_batch == 2, "two workers per batch"
    vtiles = (vocab + 127) // 128

    @functools.partial(
        pl.kernel,
        mesh=plsc.VectorSubcoreMesh(core_axis_name="c", subcore_axis_name="s"),
        compiler_params=pltpu.CompilerParams(
            needs_layout_passes=False,
            disable_bounds_checks=True,
            disable_semaphore_checks=True,
        ),
        out_type=jax.ShapeDtypeStruct(
            (n_batch, n_blocks, vtiles * 128), jnp.float32
        ),
        scratch_types=[
            pltpu.VMEM((bpw * _BLOCK,), jnp.int32),
            pltpu.VMEM((2, vtiles, 8, 128), jnp.float32),
            pltpu.SemaphoreType.DMA,
        ],
    )
    def hist_k(tok_hbm, hist_hbm, tok_v, acc_v, sem):
        wid = lax.axis_index("s") * nc + lax.axis_index("c")
        b = wid // 2
        h = wid % 2
        in_cp = pltpu.async_copy(
            tok_hbm.at[pl.ds(wid * (bpw * _BLOCK), bpw * _BLOCK)], tok_v, sem
        )

        zeros = jnp.zeros((lanes,), jnp.float32)

        def zero_body(i, carry):
            c = i // 8
            r = i % 8
            for g in range(2):
                for j in range(128 // lanes):
                    acc_v[g, c, r, pl.ds(j * lanes, lanes)] = zeros
            return carry

        lax.fori_loop(0, vtiles * 8, zero_body, None)
        in_cp.wait()

        lane = lax.iota(jnp.int32, lanes)
        col_base = lane * _BLOCK
        g_idx = lane >> 3
        r_idx = lane & 7
        ones = jnp.ones((lanes,), jnp.float32)
        sunroll = 4

        def scat_body(i, carry):
            t0 = i * sunroll
            for j in range(sunroll):
                tok = plsc.load_gather(tok_v, [col_base + (t0 + j)])
                plsc.addupdate_scatter(
                    acc_v, [g_idx, tok >> 7, r_idx, tok & 127], ones
                )
            return carry

        lax.fori_loop(0, _BLOCK // sunroll, scat_body, None)

        copies = []
        for g in range(2):
            for c in range(vtiles):
                copies.append(
                    pltpu.async_copy(
                        acc_v.at[g, c],
                        hist_hbm.at[b, pl.ds(h * 16 + g * 8, 8), pl.ds(c * 128, 128)],
                        sem,
                    )
                )
        for cp in copies:
            cp.wait()

    return hist_k


def kernel(tokens, cat_embed_f, num_embed_f):
    B, L = tokens.shape
    vocab = num_embed_f.shape[0]
    n_blocks = L // _BLOCK
    hist_padded = _make_hist_kernel(B, n_blocks, vocab)(tokens.reshape(-1))
    hist = hist_padded[:, :, :vocab]
    cat_ids = tokens[:, ::_BLOCK]
    new_tokens = jnp.concatenate([cat_ids, tokens], axis=1)
    return (new_tokens, cat_ids, hist)


# speedup vs baseline: 1.3748x; 1.0020x over previous
import functools

import jax
import jax.numpy as jnp
from jax import lax
from jax.experimental import pallas as pl
from jax.experimental.pallas import tpu as pltpu
from jax.experimental.pallas import tpu_sc as plsc

_BLOCK = 64


@functools.lru_cache(maxsize=None)
def _make_hist_kernel(n_batch, n_blocks, vocab):
    info = plsc.get_sparse_core_info()
    nc, ns, lanes = info.num_cores, info.num_subcores, info.num_lanes
    nw = nc * ns
    total_blocks = n_batch * n_blocks
    assert total_blocks % nw == 0
    bpw = total_blocks // nw
    assert bpw == lanes == 16, "one block per vreg lane"
    assert nw % n_batch == 0 and nw // n_batch == 2, "two workers per batch"
    vtiles = (vocab + 127) // 128

    @functools.partial(
        pl.kernel,
        mesh=plsc.VectorSubcoreMesh(core_axis_name="c", subcore_axis_name="s"),
        compiler_params=pltpu.CompilerParams(
            needs_layout_passes=False,
            disable_bounds_checks=True,
            disable_semaphore_checks=True,
            skip_device_barrier=True,
        ),
        out_type=jax.ShapeDtypeStruct(
            (n_batch, n_blocks, vtiles * 128), jnp.float32
        ),
        scratch_types=[
            pltpu.VMEM((bpw * _BLOCK,), jnp.int32),
            pltpu.VMEM((2, vtiles, 8, 128), jnp.float32),
            pltpu.SemaphoreType.DMA,
        ],
    )
    def hist_k(tok_hbm, hist_hbm, tok_v, acc_v, sem):
        wid = lax.axis_index("s") * nc + lax.axis_index("c")
        b = wid // 2
        h = wid % 2
        in_cp = pltpu.async_copy(
            tok_hbm.at[pl.ds(wid * (bpw * _BLOCK), bpw * _BLOCK)], tok_v, sem
        )

        zeros = jnp.zeros((lanes,), jnp.float32)

        def zero_body(i, carry):
            c = i // 8
            r = i % 8
            for g in range(2):
                for j in range(128 // lanes):
                    acc_v[g, c, r, pl.ds(j * lanes, lanes)] = zeros
            return carry

        lax.fori_loop(0, vtiles * 8, zero_body, None)
        in_cp.wait()

        lane = lax.iota(jnp.int32, lanes)
        col_base = lane * _BLOCK
        g_idx = lane >> 3
        r_idx = lane & 7
        ones = jnp.ones((lanes,), jnp.float32)
        sunroll = 4

        def scat_body(i, carry):
            t0 = i * sunroll
            for j in range(sunroll):
                tok = plsc.load_gather(tok_v, [col_base + (t0 + j)])
                plsc.addupdate_scatter(
                    acc_v, [g_idx, tok >> 7, r_idx, tok & 127], ones
                )
            return carry

        lax.fori_loop(0, _BLOCK // sunroll, scat_body, None)

        copies = []
        for g in range(2):
            for c in range(vtiles):
                copies.append(
                    pltpu.async_copy(
                        acc_v.at[g, c],
                        hist_hbm.at[b, pl.ds(h * 16 + g * 8, 8), pl.ds(c * 128, 128)],
                        sem,
                    )
                )
        for cp in copies:
            cp.wait()

    return hist_k


def kernel(tokens, cat_embed_f, num_embed_f):
    B, L = tokens.shape
    vocab = num_embed_f.shape[0]
    n_blocks = L // _BLOCK
    hist_padded = _make_hist_kernel(B, n_blocks, vocab)(tokens.reshape(-1))
    hist = hist_padded[:, :, :vocab]
    cat_ids = tokens[:, ::_BLOCK]
    new_tokens = jnp.concatenate([cat_ids, tokens], axis=1)
    return (new_tokens, cat_ids, hist)
